# baseline (speedup 1.0000x reference)
import functools

import jax
import jax.numpy as jnp
import numpy as np
from jax import lax
from jax.experimental import pallas as pl
from jax.experimental.pallas import tpu as pltpu
from jax.experimental.pallas import tpu_sc as plsc

D = 768
DFF = 3072
G = 4
EG = 2
E = G * EG
TOPK = 2
N = 2048

BT = 256
NJ = N // BT
CAP = E * N
CAPP = CAP + BT
NW = 32
TPB = N // NW
SPW = CAP // NW

_SWAP = np.zeros((E, E), np.float32)
for _i in range(E):
    _SWAP[_i ^ 1, _i] = 1.0
_EXPAND = np.zeros((G, E), np.float32)
for _g in range(G):
    _EXPAND[_g, 2 * _g] = 1.0
    _EXPAND[_g, 2 * _g + 1] = 1.0


def _precise_div(a, b):
    r = 1.0 / b
    r = r * (2.0 - b * r)
    r = r * (2.0 - b * r)
    q = a * r
    q = q + (a - q * b) * r
    return q


def _router_kernel(x_ref, wg_ref, we_ref, swap_ref, expand_ref,
                   wt_ref, st_ref, cnt_ref):
    x = x_ref[...]
    gl = lax.dot_general(x, wg_ref[...], (((1,), (1,)), ((), ())),
                         preferred_element_type=jnp.float32)
    gmax = jnp.max(gl, axis=-1, keepdims=True)
    gexp = jnp.exp(gl - gmax)
    gp = _precise_div(gexp, jnp.sum(gexp, axis=-1, keepdims=True))
    el = lax.dot_general(x, we_ref[...], (((1,), (1,)), ((), ())),
                         preferred_element_type=jnp.float32)
    swap = swap_ref[...]
    partner = lax.dot_general(el, swap, (((1,), (0,)), ((), ())),
                              preferred_element_type=jnp.float32,
                              precision=lax.Precision.HIGHEST)
    emax = jnp.maximum(el, partner)
    eexp = jnp.exp(el - emax)
    pexp = lax.dot_general(eexp, swap, (((1,), (0,)), ((), ())),
                           preferred_element_type=jnp.float32,
                           precision=lax.Precision.HIGHEST)
    ep = _precise_div(eexp, eexp + pexp)
    gp8 = lax.dot_general(gp, expand_ref[...], (((1,), (0,)), ((), ())),
                          preferred_element_type=jnp.float32,
                          precision=lax.Precision.HIGHEST)
    valid = (jnp.where(gp8 >= (1.0 / G), 1.0, 0.0)
             * jnp.where(ep >= (1.0 / EG), 1.0, 0.0))
    fp = gp8 * ep
    nsel = jnp.sum(valid, axis=-1, keepdims=True)
    lanes = lax.broadcasted_iota(jnp.int32, fp.shape, 1)
    m1 = jnp.max(fp, axis=-1, keepdims=True)
    i1 = jnp.min(jnp.where(fp == m1, lanes, E), axis=-1, keepdims=True)
    fp2 = jnp.where(lanes == i1, -1.0, fp)
    m2 = jnp.max(fp2, axis=-1, keepdims=True)
    i2 = jnp.min(jnp.where(fp2 == m2, lanes, E), axis=-1, keepdims=True)
    topk_mask = (jnp.where(lanes == i1, 1.0, 0.0)
                 + jnp.where(lanes == i2, 1.0, 0.0))
    final_mask = jnp.where(nsel < TOPK, topk_mask, valid)
    sel_w = fp * final_mask
    wsum = jnp.maximum(jnp.sum(sel_w, axis=-1, keepdims=True), 1e-9)
    w = _precise_div(sel_w, wsum)
    wt_ref[...] = w.T
    r_iota = lax.broadcasted_iota(jnp.int32, (N, N), 0)
    c_iota = lax.broadcasted_iota(jnp.int32, (N, N), 1)
    tril = jnp.where(c_iota < r_iota, 1.0, 0.0)
    rank = lax.dot_general(tril, final_mask, (((1,), (0,)), ((), ())),
                           preferred_element_type=jnp.float32)
    elane = lax.broadcasted_iota(jnp.int32, (N, E), 1).astype(jnp.float32)
    slots = jnp.where(final_mask > 0, elane * N + rank, float(CAP))
    st_ref[...] = slots.T.astype(jnp.int32)
    cnt_ref[...] = jnp.sum(final_mask, axis=0, keepdims=True).astype(jnp.int32)


_SC_MESH = plsc.VectorSubcoreMesh(core_axis_name="c", subcore_axis_name="s")


def _wid():
    return lax.axis_index("s") * 2 + lax.axis_index("c")


def _scatter_body(st_hbm, wt_hbm, tok_hbm, wslot_hbm, slots_v, w_v, tokv, sem):
    base = _wid() * TPB
    for e in range(E):
        pltpu.sync_copy(st_hbm.at[e, pl.ds(base, TPB)], slots_v.at[e])
        pltpu.sync_copy(wt_hbm.at[e, pl.ds(base, TPB)], w_v.at[e])
    for v in range(TPB // 16):
        tokv[pl.ds(16 * v, 16)] = lax.iota(jnp.int32, 16) + base + 16 * v
    handles = []
    for e in range(E):
        for v in range(TPB // 16):
            sl = slots_v[e, pl.ds(16 * v, 16)]
            handles.append(pltpu.make_async_copy(
                tokv.at[pl.ds(16 * v, 16)], tok_hbm.at[sl], sem))
            handles.append(pltpu.make_async_copy(
                w_v.at[e, pl.ds(16 * v, 16)], wslot_hbm.at[sl], sem))
    for h in handles:
        h.start()
    for h in handles:
        h.wait()


def _gather_body(tok_hbm, x_hbm, xg_hbm, tokv, rows_v, sem):
    base = _wid() * SPW
    for ch in range(SPW // 16):
        r0 = base + ch * 16
        pltpu.sync_copy(tok_hbm.at[pl.ds(r0, 16)], tokv)
        t = jnp.minimum(jnp.maximum(tokv[...], 0), N - 1)
        pltpu.async_copy(x_hbm.at[t], rows_v, sem).wait()
        pltpu.sync_copy(rows_v, xg_hbm.at[pl.ds(r0, 16)])


def _combine_body(st_hbm, yg_hbm, out_hbm, slots_v, gbuf, acc, sem):
    base = _wid() * TPB
    for e in range(E):
        pltpu.sync_copy(st_hbm.at[e, pl.ds(base, TPB)], slots_v.at[e])

    for e in range(E):
        pltpu.async_copy(yg_hbm.at[slots_v.at[e]], gbuf, sem).wait()

        def _row(r, _):
            for cc in range(D // 16):
                g = gbuf[r, pl.ds(cc * 16, 16)]
                if e == 0:
                    acc[r, pl.ds(cc * 16, 16)] = g
                else:
                    plsc.addupdate(acc.at[r, pl.ds(cc * 16, 16)], g)
            return 0

        lax.fori_loop(0, TPB, _row, 0)
    pltpu.sync_copy(acc, out_hbm.at[pl.ds(base, TPB)])


def _ffn_kernel(cnt_ref, xg_ref, w1_ref, b1_ref, w2_ref, b2_ref, ws_ref,
                yg_ref):
    e = pl.program_id(0)
    j = pl.program_id(1)
    ce = cnt_ref[jnp.minimum(e, E - 1)]
    active = jnp.logical_and(e < E, j * BT < ce)

    @pl.when(active)
    def _compute():
        xb = xg_ref[...]
        w1e = w1_ref[0]
        h = lax.dot_general(xb, w1e, (((1,), (1,)), ((), ())),
                            preferred_element_type=jnp.float32)
        h = h + b1_ref[pl.ds(jnp.minimum(e, E - 1), 1), :]
        h = 0.5 * h * (1.0 + lax.erf(h * np.float32(1.0 / np.sqrt(2.0))))
        w2e = w2_ref[0]
        o = lax.dot_general(h, w2e, (((1,), (1,)), ((), ())),
                            preferred_element_type=jnp.float32)
        o = o + b2_ref[pl.ds(jnp.minimum(e, E - 1), 1), :]
        yg_ref[...] = o * ws_ref[...]

    @pl.when(jnp.logical_not(active))
    def _zero():
        yg_ref[...] = jnp.zeros((BT, D), jnp.float32)


def _clamped_tile(e, j, cnt):
    ee = jnp.minimum(e, E - 1)
    c = cnt[ee]
    jmax = jnp.maximum((c + BT - 1) // BT - 1, 0)
    return ee * NJ + jnp.minimum(j, jmax)


@jax.jit
def kernel(x, Wg, We, w1, b1, w2, b2):
    wt, st, cnt2d = pl.pallas_call(
        _router_kernel,
        out_shape=[jax.ShapeDtypeStruct((E, N), jnp.float32),
                   jax.ShapeDtypeStruct((E, N), jnp.int32),
                   jax.ShapeDtypeStruct((1, E), jnp.int32)],
    )(x, Wg, We, jnp.asarray(_SWAP), jnp.asarray(_EXPAND))
    cnt = cnt2d[0]

    scatter = pl.kernel(
        _scatter_body,
        out_type=[jax.ShapeDtypeStruct((CAPP,), jnp.int32),
                  jax.ShapeDtypeStruct((CAPP,), jnp.float32)],
        mesh=_SC_MESH,
        scratch_types=[pltpu.VMEM((E, TPB), jnp.int32),
                       pltpu.VMEM((E, TPB), jnp.float32),
                       pltpu.VMEM((TPB,), jnp.int32),
                       pltpu.SemaphoreType.DMA],
    )
    tok, wslot = scatter(st, wt)

    gather = pl.kernel(
        _gather_body,
        out_type=jax.ShapeDtypeStruct((CAPP, D), jnp.float32),
        mesh=_SC_MESH,
        scratch_types=[pltpu.VMEM((16,), jnp.int32),
                       pltpu.VMEM((16, D), jnp.float32),
                       pltpu.SemaphoreType.DMA],
    )
    xg = gather(tok, x)

    yg = pl.pallas_call(
        _ffn_kernel,
        grid_spec=pltpu.PrefetchScalarGridSpec(
            num_scalar_prefetch=1,
            grid=(E + 1, NJ),
            in_specs=[
                pl.BlockSpec((BT, D), lambda e, j, cnt: (_clamped_tile(e, j, cnt), 0)),
                pl.BlockSpec((1, DFF, D), lambda e, j, cnt: (jnp.minimum(e, E - 1), 0, 0)),
                pl.BlockSpec((E, DFF), lambda e, j, cnt: (0, 0)),
                pl.BlockSpec((1, D, DFF), lambda e, j, cnt: (jnp.minimum(e, E - 1), 0, 0)),
                pl.BlockSpec((E, D), lambda e, j, cnt: (0, 0)),
                pl.BlockSpec((BT, 1), lambda e, j, cnt: (_clamped_tile(e, j, cnt), 0)),
            ],
            out_specs=pl.BlockSpec(
                (BT, D),
                lambda e, j, cnt: (jnp.where(
                    jnp.logical_and(e < E, j * BT < cnt[jnp.minimum(e, E - 1)]),
                    e * NJ + j, CAP // BT), 0)),
        ),
        out_shape=jax.ShapeDtypeStruct((CAPP, D), jnp.float32),
    )(cnt, xg, w1, b1, w2, b2, wslot.reshape(CAPP, 1))

    combine = pl.kernel(
        _combine_body,
        out_type=jax.ShapeDtypeStruct((N, D), jnp.float32),
        mesh=_SC_MESH,
        scratch_types=[pltpu.VMEM((E, TPB), jnp.int32),
                       pltpu.VMEM((TPB, D), jnp.float32),
                       pltpu.VMEM((TPB, D), jnp.float32),
                       pltpu.SemaphoreType.DMA],
    )
    return combine(st, yg)


# speedup vs baseline: 13.7685x; 13.7685x over previous
import functools

import jax
import jax.numpy as jnp
import numpy as np
from jax import lax
from jax.experimental import pallas as pl
from jax.experimental.pallas import tpu as pltpu

D = 768
DFF = 3072
G = 4
EG = 2
E = G * EG
TOPK = 2
N = 2048

BT = 512

_SWAP = np.zeros((E, E), np.float32)
for _i in range(E):
    _SWAP[_i ^ 1, _i] = 1.0
_EXPAND = np.zeros((G, E), np.float32)
for _g in range(G):
    _EXPAND[_g, 2 * _g] = 1.0
    _EXPAND[_g, 2 * _g + 1] = 1.0


def _precise_div(a, b):
    r = 1.0 / b
    r = r * (2.0 - b * r)
    r = r * (2.0 - b * r)
    q = a * r
    q = q + (a - q * b) * r
    return q


def _router_kernel(x_ref, wg_ref, we_ref, swap_ref, expand_ref, out_ref):
    x = x_ref[...]
    gl = lax.dot_general(x, wg_ref[...], (((1,), (1,)), ((), ())),
                         preferred_element_type=jnp.float32)
    gmax = jnp.max(gl, axis=-1, keepdims=True)
    gexp = jnp.exp(gl - gmax)
    gp = _precise_div(gexp, jnp.sum(gexp, axis=-1, keepdims=True))
    el = lax.dot_general(x, we_ref[...], (((1,), (1,)), ((), ())),
                         preferred_element_type=jnp.float32)
    swap = swap_ref[...]
    partner = lax.dot_general(el, swap, (((1,), (0,)), ((), ())),
                              preferred_element_type=jnp.float32,
                              precision=lax.Precision.HIGHEST)
    emax = jnp.maximum(el, partner)
    eexp = jnp.exp(el - emax)
    pexp = lax.dot_general(eexp, swap, (((1,), (0,)), ((), ())),
                           preferred_element_type=jnp.float32,
                           precision=lax.Precision.HIGHEST)
    ep = _precise_div(eexp, eexp + pexp)
    gp8 = lax.dot_general(gp, expand_ref[...], (((1,), (0,)), ((), ())),
                          preferred_element_type=jnp.float32,
                         precision=lax.Precision.HIGHEST)
    valid = (jnp.where(gp8 >= (1.0 / G), 1.0, 0.0)
             * jnp.where(ep >= (1.0 / EG), 1.0, 0.0))
    fp = gp8 * ep
    nsel = jnp.sum(valid, axis=-1, keepdims=True)
    lanes = lax.broadcasted_iota(jnp.int32, fp.shape, 1)
    m1 = jnp.max(fp, axis=-1, keepdims=True)
    i1 = jnp.min(jnp.where(fp == m1, lanes, E), axis=-1, keepdims=True)
    fp2 = jnp.where(lanes == i1, -1.0, fp)
    m2 = jnp.max(fp2, axis=-1, keepdims=True)
    i2 = jnp.min(jnp.where(fp2 == m2, lanes, E), axis=-1, keepdims=True)
    topk_mask = (jnp.where(lanes == i1, 1.0, 0.0)
                 + jnp.where(lanes == i2, 1.0, 0.0))
    final_mask = jnp.where(nsel < TOPK, topk_mask, valid)
    sel_w = fp * final_mask
    wsum = jnp.maximum(jnp.sum(sel_w, axis=-1, keepdims=True), 1e-9)
    out_ref[...] = _precise_div(sel_w, wsum)


def _ffn_kernel(x_ref, w1_ref, b1_ref, w2_ref, b2_ref, wts_ref, out_ref):
    e = pl.program_id(0)
    j = pl.program_id(1)
    xb = x_ref[pl.ds(j * BT, BT), :]
    w1e = w1_ref[0]
    h = lax.dot_general(xb, w1e, (((1,), (1,)), ((), ())),
                        preferred_element_type=jnp.float32)
    h = h + b1_ref[pl.ds(e, 1), :]
    h = 0.5 * h * (1.0 + lax.erf(h * np.float32(1.0 / np.sqrt(2.0))))
    w2e = w2_ref[0]
    o = lax.dot_general(h, w2e, (((1,), (1,)), ((), ())),
                        preferred_element_type=jnp.float32)
    o = o + b2_ref[pl.ds(e, 1), :]
    lanes = lax.broadcasted_iota(jnp.int32, wts_ref.shape, 1)
    wcol = jnp.sum(jnp.where(lanes == e, wts_ref[...], 0.0),
                   axis=-1, keepdims=True)
    contrib = o * wcol

    @pl.when(e == 0)
    def _init():
        out_ref[pl.ds(j * BT, BT), :] = contrib

    @pl.when(e != 0)
    def _acc():
        out_ref[pl.ds(j * BT, BT), :] += contrib


@jax.jit
def kernel(x, Wg, We, w1, b1, w2, b2):
    weights = pl.pallas_call(
        _router_kernel,
        out_shape=jax.ShapeDtypeStruct((N, E), jnp.float32),
    )(x, Wg, We, jnp.asarray(_SWAP), jnp.asarray(_EXPAND))

    out = pl.pallas_call(
        _ffn_kernel,
        grid=(E, N // BT),
        in_specs=[
            pl.BlockSpec((N, D), lambda e, j: (0, 0)),
            pl.BlockSpec((1, DFF, D), lambda e, j: (e, 0, 0)),
            pl.BlockSpec((E, DFF), lambda e, j: (0, 0)),
            pl.BlockSpec((1, D, DFF), lambda e, j: (e, 0, 0)),
            pl.BlockSpec((E, D), lambda e, j: (0, 0)),
            pl.BlockSpec((BT, E), lambda e, j: (j, 0)),
        ],
        out_specs=pl.BlockSpec((N, D), lambda e, j: (0, 0)),
        out_shape=jax.ShapeDtypeStruct((N, D), jnp.float32),
    )(x, w1, b1, w2, b2, weights)
    return out
